# baseline (speedup 1.0000x reference)
import functools

import jax
import jax.numpy as jnp
from jax import lax
from jax.experimental import pallas as pl
from jax.experimental.pallas import tpu as pltpu
from jax.experimental.pallas import tpu_sc as plsc

_ROWS = 257
_D = 128
_NUM_CORES = 2
_NUM_SUBCORES = 16
_NW = _NUM_CORES * _NUM_SUBCORES
_RPW = 256 // _NW

_mesh = plsc.VectorSubcoreMesh(core_axis_name="c", subcore_axis_name="s")


@functools.partial(
    pl.kernel,
    mesh=_mesh,
    out_type=jax.ShapeDtypeStruct((_ROWS, _D), jnp.float32),
    scratch_types=[pltpu.VMEM((_RPW + 1, _D), jnp.float32)],
)
def _lookup(table_hbm, out_hbm, buf):
    wid = lax.axis_index("s") * _NUM_CORES + lax.axis_index("c")
    base = wid * _RPW
    pltpu.sync_copy(table_hbm.at[pl.ds(base, _RPW)], buf.at[pl.ds(0, _RPW)])
    pltpu.sync_copy(buf.at[pl.ds(0, _RPW)], out_hbm.at[pl.ds(base, _RPW)])

    @pl.when(wid == 0)
    def _tail():
        pltpu.sync_copy(table_hbm.at[pl.ds(256, 1)], buf.at[pl.ds(_RPW, 1)])
        pltpu.sync_copy(buf.at[pl.ds(_RPW, 1)], out_hbm.at[pl.ds(256, 1)])


def kernel(seq_len, table):
    del seq_len
    return _lookup(table)
